# baseline (speedup 1.0000x reference)
import functools

import jax
import jax.numpy as jnp
from jax import lax
from jax.experimental import pallas as pl
from jax.experimental.pallas import tpu as pltpu
from jax.experimental.pallas import tpu_sc as plsc

_L = 16
_CHUNK = 128


def _softplus(x):
    y = jnp.exp(-jnp.abs(x))
    s = y / (2.0 + y)
    s2 = s * s
    p = 2.0 * s * (1.0 + s2 * (1.0 / 3.0 + s2 * (0.2 + s2 * (1.0 / 7.0))))
    return jnp.maximum(x, 0.0) + p


@functools.lru_cache(maxsize=None)
def _build(B, V, D, NEG_K):
    info = plsc.get_sparse_core_info()
    NC, NS = info.num_cores, info.num_subcores
    NW = NC * NS
    assert B % (NW * _CHUNK) == 0 and NEG_K == 2
    b_per_w = B // NW
    n_sub = b_per_w // _CHUNK

    mesh = plsc.VectorSubcoreMesh(core_axis_name="c", subcore_axis_name="s")

    @functools.partial(
        pl.kernel,
        mesh=mesh,
        compiler_params=pltpu.CompilerParams(
            needs_layout_passes=False, use_tc_tiling_on_sc=False),
        out_type=jax.ShapeDtypeStruct((B,), jnp.float32),
        scratch_types=[
            pltpu.VMEM((_CHUNK,), jnp.int32),
            pltpu.VMEM((_CHUNK,), jnp.int32),
            pltpu.VMEM((_CHUNK,), jnp.int32),
            pltpu.VMEM((_CHUNK,), jnp.int32),
            pltpu.VMEM((_CHUNK, D), jnp.float32),
            pltpu.VMEM((_CHUNK, D), jnp.float32),
            pltpu.VMEM((_CHUNK, D), jnp.float32),
            pltpu.VMEM((_CHUNK, D), jnp.float32),
            pltpu.VMEM((_CHUNK * _L,), jnp.float32),
            pltpu.VMEM((_CHUNK * _L,), jnp.float32),
            pltpu.VMEM((_CHUNK * _L,), jnp.float32),
            pltpu.VMEM((_CHUNK,), jnp.float32),
            pltpu.SemaphoreType.DMA,
        ],
    )
    def line_sc(i_hbm, j_hbm, neg_hbm, emb_hbm, out_hbm,
                idx_i, idx_j, idx_n0, idx_n1, ri, rj, rn0, rn1,
                part_p, part_0, part_1, out_v, sem):
        wid = lax.axis_index("s") * NC + lax.axis_index("c")
        lanes = lax.iota(jnp.int32, 16)
        zero = jnp.zeros((_L,), jnp.float32)
        for sub in range(n_sub):
            base = wid * b_per_w + sub * _CHUNK
            pltpu.sync_copy(i_hbm.at[pl.ds(base, _CHUNK)], idx_i)
            pltpu.sync_copy(j_hbm.at[pl.ds(base, _CHUNK)], idx_j)
            pltpu.sync_copy(neg_hbm.at[0, pl.ds(base, _CHUNK)], idx_n0)
            pltpu.sync_copy(neg_hbm.at[1, pl.ds(base, _CHUNK)], idx_n1)
            copies = [
                pltpu.make_async_copy(emb_hbm.at[idx_i], ri, sem),
                pltpu.make_async_copy(emb_hbm.at[idx_j], rj, sem),
                pltpu.make_async_copy(emb_hbm.at[idx_n0], rn0, sem),
                pltpu.make_async_copy(emb_hbm.at[idx_n1], rn1, sem),
            ]
            for cp in copies:
                cp.start()
            for cp in copies:
                cp.wait()

            def row_body(r, _):
                vi = [ri[r, pl.ds(c * _L, _L)] for c in range(D // _L)]
                vj = [rj[r, pl.ds(c * _L, _L)] for c in range(D // _L)]
                v0 = [rn0[r, pl.ds(c * _L, _L)] for c in range(D // _L)]
                v1 = [rn1[r, pl.ds(c * _L, _L)] for c in range(D // _L)]
                pp = zero
                p0 = zero
                p1 = zero
                for c in range(D // _L):
                    pp = pp + vi[c] * vj[c]
                    p0 = p0 + vi[c] * v0[c]
                    p1 = p1 + vi[c] * v1[c]
                part_p[pl.ds(r * _L, _L)] = pp
                part_0[pl.ds(r * _L, _L)] = p0
                part_1[pl.ds(r * _L, _L)] = p1
                return 0

            lax.fori_loop(0, _CHUNK, row_body, 0)

            def group_body(g, _):
                base_idx = (g * _L + lanes) * _L
                ap, a0, a1 = zero, zero, zero
                for k in range(_L):
                    ap = ap + plsc.load_gather(part_p, [base_idx + k])
                    a0 = a0 + plsc.load_gather(part_0, [base_idx + k])
                    a1 = a1 + plsc.load_gather(part_1, [base_idx + k])
                res = _softplus(-ap) + _softplus(a0) + _softplus(a1)
                out_v[pl.ds(g * _L, _L)] = res
                return 0

            lax.fori_loop(0, _CHUNK // _L, group_body, 0)
            pltpu.sync_copy(out_v, out_hbm.at[pl.ds(base, _CHUNK)])

    return line_sc


def kernel(i, j, neg_set, emb):
    B = i.shape[0]
    V, D = emb.shape
    fn = _build(B, V, D, neg_set.shape[0])
    return fn(i.astype(jnp.int32), j.astype(jnp.int32),
              neg_set.astype(jnp.int32), emb)
